# Initial kernel scaffold; baseline (speedup 1.0000x reference)
import jax
import jax.numpy as jnp
from jax.experimental import pallas as pl


def kernel(iitem, oitems, nitems, W_i, W_o):
    raise NotImplementedError("write your pallas kernel here")


# speedup vs baseline: 5.4021x; 5.4021x over previous
import functools

import jax
import jax.numpy as jnp
from jax import lax
from jax.experimental import pallas as pl
from jax.experimental.pallas import tpu as pltpu
from jax.experimental.pallas import tpu_sc as plsc

_L = 16


def _sc_dots(W_i, W_o, idx_all, iidx, n_pad):
    B = iidx.shape[0]
    V, D = W_o.shape
    info = plsc.get_sparse_core_info()
    nw = info.num_cores * info.num_subcores
    bpw = B // nw
    n_vec = D // _L

    mesh = plsc.VectorSubcoreMesh(core_axis_name="c", subcore_axis_name="s")

    @functools.partial(
        pl.kernel,
        out_type=jax.ShapeDtypeStruct((B, n_pad), jnp.float32),
        mesh=mesh,
        scratch_types=[
            pltpu.VMEM((bpw,), jnp.int32),
            pltpu.VMEM((bpw, D), jnp.float32),
            pltpu.VMEM((n_pad,), jnp.int32),
            pltpu.VMEM((n_pad, D), jnp.float32),
            pltpu.VMEM((n_pad,), jnp.float32),
            pltpu.VMEM((17 * _L,), jnp.float32),
            pltpu.SemaphoreType.DMA,
        ],
        compiler_params=pltpu.CompilerParams(needs_layout_passes=False,
                                             use_tc_tiling_on_sc=False),
    )
    def k(wi_h, wo_h, idx_h, iidx_h, out_h, iidx_v, ivecs, idx_v, rows, dots,
          tsc, sem):
        wid = lax.axis_index("s") * info.num_cores + lax.axis_index("c")
        base = wid * bpw
        pltpu.sync_copy(iidx_h.at[pl.ds(base, bpw)], iidx_v)
        pltpu.async_copy(wi_h.at[iidx_v], ivecs, sem).wait()

        def batch_body(b, carry):
            row = base + b
            pltpu.sync_copy(idx_h.at[row], idx_v)
            pltpu.async_copy(wo_h.at[idx_v], rows, sem).wait()
            iv = [ivecs[b, pl.ds(_L * j, _L)] for j in range(n_vec)]

            lanes17 = lax.iota(jnp.int32, _L) * 17

            def group_body(g, c2):
                r0 = g * _L
                for u in range(_L):
                    r = r0 + u
                    p = rows[r, pl.ds(0, _L)] * iv[0]
                    for j in range(1, n_vec):
                        p = p + rows[r, pl.ds(_L * j, _L)] * iv[j]
                    tsc[pl.ds(17 * u, _L)] = p
                acc = plsc.load_gather(tsc, [lanes17])
                for c in range(1, _L):
                    acc = acc + plsc.load_gather(tsc, [lanes17 + c])
                dots[pl.ds(r0, _L)] = acc
                return c2

            lax.fori_loop(0, n_pad // _L, group_body, 0, unroll=False)
            pltpu.sync_copy(dots, out_h.at[row])
            return carry

        lax.fori_loop(0, bpw, batch_body, 0, unroll=False)

    return k(W_i, W_o, idx_all, iidx)


def _tc_loss(dots, n_ctx, n_valid):
    B, n_pad = dots.shape

    def body(d_ref, o_ref):
        x = d_ref[...]
        col = lax.broadcasted_iota(jnp.int32, (B, n_pad), 1)
        t = jnp.where(col < n_ctx, x, -x)
        ls = jnp.minimum(t, 0.0) - jnp.log1p(jnp.exp(-jnp.abs(t)))
        ls = jnp.where(col < n_valid, ls, 0.0)
        o_ref[0, 0] = -jnp.sum(ls) / B

    out = pl.pallas_call(
        body,
        out_shape=jax.ShapeDtypeStruct((1, 1), jnp.float32),
        out_specs=pl.BlockSpec(memory_space=pltpu.SMEM),
    )(dots)
    return out[0, 0]


def kernel(iitem, oitems, nitems, W_i, W_o):
    B, C = oitems.shape
    n_valid = C + nitems.shape[1]
    n_pad = -(-n_valid // _L) * _L
    idx = jnp.concatenate([oitems, nitems], axis=1).astype(jnp.int32)
    idx = jnp.pad(idx, ((0, 0), (0, n_pad - n_valid)))
    dots = _sc_dots(W_i, W_o, idx, iitem.astype(jnp.int32), n_pad)
    return _tc_loss(dots, C, n_valid)
